# Initial kernel scaffold; baseline (speedup 1.0000x reference)
import jax
import jax.numpy as jnp
from jax.experimental import pallas as pl


def kernel(x, edge_index, W1, b1, g1, be1, W2, b2, gO, bO, eps):
    raise NotImplementedError("write your pallas kernel here")


# speedup vs baseline: 7.1907x; 7.1907x over previous
import functools

import jax
import jax.numpy as jnp
from jax import lax
from jax.experimental import pallas as pl
from jax.experimental.pallas import tpu as pltpu
from jax.experimental.pallas import tpu_sc as plsc

N = 10000
E = 320000
D = 128
L = 2
BN_EPS = 1e-5

NC = 2
NS = 16
CHUNK = 80
EDGES_PER_TILE = E // (NC * NS)
CHUNKS_PER_TILE = EDGES_PER_TILE // CHUNK
N_PAD = 10240
ROWS_PER_TILE = N_PAD // NS


def _spmm_body(src_hbm, dst_hbm, h_hbm, zeros_hbm, out_hbm,
               src_v, dst_v, rows_v, pooled_sh, sem):
    c = lax.axis_index("c")
    s = lax.axis_index("s")
    wid = c * NS + s

    pltpu.sync_copy(zeros_hbm.at[pl.ds(s * ROWS_PER_TILE, ROWS_PER_TILE)],
                    pooled_sh.at[pl.ds(s * ROWS_PER_TILE, ROWS_PER_TILE)])

    pltpu.sync_copy(src_hbm.at[wid], src_v)
    pltpu.sync_copy(dst_hbm.at[wid], dst_v)

    plsc.subcore_barrier()

    def body(i, carry):
        pltpu.async_copy(h_hbm.at[src_v.at[i]], rows_v, sem).wait()
        pltpu.sync_copy(rows_v, pooled_sh.at[dst_v.at[i]], add=True)
        return carry

    lax.fori_loop(0, CHUNKS_PER_TILE, body, 0)

    plsc.subcore_barrier()

    pltpu.sync_copy(pooled_sh.at[pl.ds(s * ROWS_PER_TILE, ROWS_PER_TILE)],
                    out_hbm.at[c].at[pl.ds(s * ROWS_PER_TILE, ROWS_PER_TILE)])


@jax.jit
def _spmm_partials(src2d, dst2d, h, zeros):
    mesh = plsc.VectorSubcoreMesh(core_axis_name="c", subcore_axis_name="s")
    k = pl.kernel(
        _spmm_body,
        mesh=mesh,
        out_type=jax.ShapeDtypeStruct((NC, N_PAD, D), jnp.float32),
        scratch_types=[
            pltpu.VMEM((CHUNKS_PER_TILE, CHUNK), jnp.int32),
            pltpu.VMEM((CHUNKS_PER_TILE, CHUNK), jnp.int32),
            pltpu.VMEM((CHUNK, D), jnp.float32),
            pltpu.VMEM_SHARED((N_PAD, D), jnp.float32),
            pltpu.SemaphoreType.DMA,
        ],
    )
    return k(src2d, dst2d, h, zeros)


def _mlp_body(pp_ref, h_ref, w1_ref, b1_ref, g1_ref, be1_ref,
              w2_ref, b2_ref, gO_ref, bO_ref, eps_ref, out_ref):
    pooled = ((pp_ref[0, :N, :] + pp_ref[1, :N, :])
              + (1.0 + eps_ref[0, 0]) * h_ref[...])
    a = lax.dot_general(pooled, w1_ref[...],
                        dimension_numbers=(((1,), (1,)), ((), ())),
                        preferred_element_type=jnp.float32) + b1_ref[...]
    m = jnp.mean(a, axis=0, keepdims=True)
    v = jnp.mean((a - m) * (a - m), axis=0, keepdims=True)
    h1 = jnp.maximum(
        (a - m) * lax.rsqrt(v + BN_EPS) * g1_ref[...] + be1_ref[...], 0.0)
    o = lax.dot_general(h1, w2_ref[...],
                        dimension_numbers=(((1,), (1,)), ((), ())),
                        preferred_element_type=jnp.float32) + b2_ref[...]
    m2 = jnp.mean(o, axis=0, keepdims=True)
    v2 = jnp.mean((o - m2) * (o - m2), axis=0, keepdims=True)
    out_ref[...] = jnp.maximum(
        (o - m2) * lax.rsqrt(v2 + BN_EPS) * gO_ref[...] + bO_ref[...], 0.0)


@jax.jit
def _mlp(pp, h, w1, b1, g1, be1, w2, b2, gO, bO, eps_l):
    return pl.pallas_call(
        _mlp_body,
        out_shape=jax.ShapeDtypeStruct((N, D), jnp.float32),
    )(pp, h, w1, b1.reshape(1, D), g1.reshape(1, D), be1.reshape(1, D),
      w2, b2.reshape(1, D), gO.reshape(1, D), bO.reshape(1, D),
      eps_l.reshape(1, 1))


def kernel(x, edge_index, W1, b1, g1, be1, W2, b2, gO, bO, eps):
    dst = edge_index[0]
    src = edge_index[1]
    src3d = src.reshape(NC * NS, CHUNKS_PER_TILE, CHUNK)
    dst3d = dst.reshape(NC * NS, CHUNKS_PER_TILE, CHUNK)
    zeros = jnp.zeros((N_PAD, D), jnp.float32)
    h = x
    for l in range(L):
        pp = _spmm_partials(src3d, dst3d, h, zeros)
        h = _mlp(pp, h, W1[l], b1[l], g1[l], be1[l],
                 W2[l], b2[l], gO[l], bO[l], eps[l])
    return h
